# Initial kernel scaffold; baseline (speedup 1.0000x reference)
import jax
import jax.numpy as jnp
from jax.experimental import pallas as pl


def kernel(x, style_fm_matched, tight_mask, loss_mask):
    raise NotImplementedError("write your pallas kernel here")


# speedup vs baseline: 2963.2060x; 2963.2060x over previous
import functools

import jax
import jax.numpy as jnp
from jax.experimental import pallas as pl

_NB = 256
_F32 = jnp.float32


def _fiota(shape, dim):
    return jax.lax.broadcasted_iota(jnp.int32, shape, dim).astype(_F32)


def _minmax_body(v_ref, m_ref, lo_ref, hi_ref):
    v = v_ref[0] * m_ref[...]
    lo_ref[0] = jnp.min(v, keepdims=True)
    hi_ref[0] = jnp.max(v, keepdims=True)


def _hist_body(v_ref, m_ref, lo_ref, hi_ref, hist_ref):
    v = v_ref[0] * m_ref[...]
    s0, s1 = v.shape
    lo = lo_ref[0, 0, 0]
    hi = hi_ref[0, 0, 0]
    width = (hi - lo) / 256.0
    b = jnp.clip(jnp.floor((v - lo) / (width + 1e-12)), 0.0, 255.0)
    bhi = jnp.floor(b * (1.0 / 16.0))
    blo = b - bhi * 16.0
    pidx = _fiota((16, s1), 0)
    hist = jnp.zeros((16, 16), _F32)
    for s in range(s0):
        hh = (bhi[s:s + 1, :] == pidx).astype(_F32)
        ll = (blo[s:s + 1, :] == pidx).astype(_F32)
        hist = hist + jax.lax.dot_general(
            hh, ll, (((1,), (1,)), ((), ())), preferred_element_type=_F32)
    hist_ref[0] = hist


def _smalls_body(hist_ref, lo_ref, hi_ref, e_ref, ct_ref, *, n):
    hist = hist_ref[...]
    cc = hist.shape[0]
    hsum = jnp.sum(hist, axis=1, keepdims=True)
    cdf = hist * n / hsum
    for s in (1, 2, 4, 8, 16, 32, 64, 128):
        pad = jnp.zeros((cc, s), _F32)
        cdf = cdf + jnp.concatenate([pad, cdf[:, :_NB - s]], axis=1)
    e_ref[...] = jnp.clip(jnp.floor(cdf), 0.0, n)
    cdf0 = cdf[0:1, :]
    prev0 = jnp.concatenate(
        [jnp.zeros((1, 1), _F32), cdf0[:, :_NB - 1]], axis=1)
    cdf1_0 = cdf[1:2, 0:1]
    cdfflat = jnp.concatenate(
        [cdf0, cdf1_0, jnp.zeros((1, 255), _F32)], axis=1)
    prevflat = jnp.concatenate(
        [prev0, jnp.zeros((1, 256), _F32)], axis=1)
    kk = 512
    rng = _fiota((kk, 1), 0) + 1.0
    idxc = jnp.sum((cdf0 < rng).astype(_F32), axis=1, keepdims=True)
    miota = _fiota((1, kk), 1)
    sel = (idxc == miota).astype(_F32)
    cdfg = jnp.sum(sel * cdfflat, axis=1, keepdims=True)
    prevg = jnp.sum(sel * prevflat, axis=1, keepdims=True)
    ratio = jnp.clip((rng - prevg) / (1e-08 + cdfg), 0.0, 1.0)
    lo_x = lo_ref[0, 0, 0]
    hi_x = hi_ref[0, 0, 0]
    step = (hi_x - lo_x) / 256.0
    ct_ref[...] = lo_x + (ratio + idxc) * step


def _xpass_body(x_ref, m_ref, et_ref, ct_ref, out_ref, *, n):
    fm = x_ref[0] * m_ref[...]
    qb = fm.shape[0]
    ecol = et_ref[0]
    ct = ct_ref[...]
    sq = jnp.sum(fm * fm)
    bsum = jnp.sum(fm, axis=1, keepdims=True)
    stot = jnp.sum(bsum)
    qe = jnp.floor(ecol * (1.0 / 128.0))
    re = ecol - qe * 128.0
    qiota = _fiota((_NB, qb), 1)
    mask1 = (qiota < qe).astype(_F32)
    term1 = jax.lax.dot_general(mask1, bsum, (((1,), (0,)), ((), ())),
                                precision=jax.lax.Precision.HIGHEST,
                                preferred_element_type=_F32)
    oneq = (qiota == qe).astype(_F32)
    g = jax.lax.dot_general(oneq, fm, (((1,), (0,)), ((), ())),
                            precision=jax.lax.Precision.HIGHEST,
                            preferred_element_type=_F32)
    riota = _fiota((_NB, 128), 1)
    mask2 = (riota < re).astype(_F32)
    term2 = jnp.sum(g * mask2, axis=1, keepdims=True)
    p = term1 + term2
    ct0 = ct[:_NB, :]
    ct1 = ct[1:_NB + 1, :]
    d = ct0 - ct1
    ct256 = jnp.sum(ct[_NB:_NB + 1, :])
    cross = jnp.sum(p * d) + ct256 * stot
    eprev = jnp.concatenate(
        [jnp.zeros((1, 1), _F32), ecol[:_NB - 1, :]], axis=0)
    cnt = ecol - eprev
    e255 = jnp.sum(ecol[_NB - 1:_NB, :])
    sqt = jnp.sum(ct0 * ct0 * cnt) + ct256 * ct256 * (n - e255)
    out_ref[0] = jnp.broadcast_to(sqt - 2.0 * cross + sq, (1, 1))


def kernel(x, style_fm_matched, tight_mask, loss_mask):
    del loss_mask
    c = x.shape[1]
    n = x.shape[2] * x.shape[3]
    nf = float(n)
    s1 = n // 16
    qb = n // 128
    xs = style_fm_matched.reshape(c, 16, s1)
    m16 = tight_mask.reshape(16, s1)
    lo_s, hi_s = pl.pallas_call(
        _minmax_body,
        grid=(c,),
        in_specs=[pl.BlockSpec((1, 16, s1), lambda i: (i, 0, 0)),
                  pl.BlockSpec((16, s1), lambda i: (0, 0))],
        out_specs=[pl.BlockSpec((1, 1, 1), lambda i: (i, 0, 0)),
                   pl.BlockSpec((1, 1, 1), lambda i: (i, 0, 0))],
        out_shape=[jax.ShapeDtypeStruct((c, 1, 1), _F32),
                   jax.ShapeDtypeStruct((c, 1, 1), _F32)],
    )(xs, m16)
    hist16 = pl.pallas_call(
        _hist_body,
        grid=(c,),
        in_specs=[pl.BlockSpec((1, 16, s1), lambda i: (i, 0, 0)),
                  pl.BlockSpec((16, s1), lambda i: (0, 0)),
                  pl.BlockSpec((1, 1, 1), lambda i: (i, 0, 0)),
                  pl.BlockSpec((1, 1, 1), lambda i: (i, 0, 0))],
        out_specs=pl.BlockSpec((1, 16, 16), lambda i: (i, 0, 0)),
        out_shape=jax.ShapeDtypeStruct((c, 16, 16), _F32),
    )(xs, m16, lo_s, hi_s)
    hist = hist16.reshape(c, _NB)
    xq = x.reshape(c, qb, 128)
    mq = tight_mask.reshape(qb, 128)
    lo_x, hi_x = pl.pallas_call(
        _minmax_body,
        grid=(1,),
        in_specs=[pl.BlockSpec((1, qb, 128), lambda i: (0, 0, 0)),
                  pl.BlockSpec((qb, 128), lambda i: (0, 0))],
        out_specs=[pl.BlockSpec((1, 1, 1), lambda i: (0, 0, 0)),
                   pl.BlockSpec((1, 1, 1), lambda i: (0, 0, 0))],
        out_shape=[jax.ShapeDtypeStruct((1, 1, 1), _F32),
                   jax.ShapeDtypeStruct((1, 1, 1), _F32)],
    )(xq, mq)
    e, ct = pl.pallas_call(
        functools.partial(_smalls_body, n=nf),
        out_shape=[jax.ShapeDtypeStruct((c, _NB), _F32),
                   jax.ShapeDtypeStruct((512, 1), _F32)],
    )(hist, lo_x, hi_x)
    e3 = e.reshape(c, _NB, 1)
    perch = pl.pallas_call(
        functools.partial(_xpass_body, n=nf),
        grid=(c,),
        in_specs=[pl.BlockSpec((1, qb, 128), lambda i: (i, 0, 0)),
                  pl.BlockSpec((qb, 128), lambda i: (0, 0)),
                  pl.BlockSpec((1, _NB, 1), lambda i: (i, 0, 0)),
                  pl.BlockSpec((512, 1), lambda i: (0, 0))],
        out_specs=pl.BlockSpec((1, 1, 1), lambda i: (i, 0, 0)),
        out_shape=jax.ShapeDtypeStruct((c, 1, 1), _F32),
    )(xq, mq, e3, ct)
    return jnp.sum(perch) / (c * nf)
